# baseline (speedup 1.0000x reference)
import functools

import jax
import jax.numpy as jnp
from jax import lax
from jax.experimental import pallas as pl
from jax.experimental.pallas import tpu as pltpu
from jax.experimental.pallas import tpu_sc as plsc

N = 2048
D = 768
E = 64
H = 1536
CAP = 256
NC, NS, L = 2, 16, 16
NW = NC * NS
EPW = E // NW
SLOTS = E * CAP
SPW = SLOTS // NW
TBLK = 256
GCH = 64

_PREC = lax.Precision.HIGHEST

_mesh = plsc.VectorSubcoreMesh(
    core_axis_name="c", subcore_axis_name="s", num_cores=NC, num_subcores=NS)


def _router_body(x_ref, wr_ref, i1_ref, i2_ref, w1_ref, w2_ref,
                 cnt_ref, ps_ref, aux_ref):
    step = pl.program_id(0)
    logits = jnp.dot(x_ref[...], wr_ref[...],
                     preferred_element_type=jnp.float32)
    m = jnp.max(logits, axis=-1, keepdims=True)
    p = jnp.exp(logits - m)
    probs = p / jnp.sum(p, axis=-1, keepdims=True)
    lane = lax.broadcasted_iota(jnp.int32, (TBLK, E), 1)
    v1 = jnp.max(probs, axis=-1, keepdims=True)
    i1 = jnp.min(jnp.where(probs == v1, lane, E), axis=-1)
    probs2 = jnp.where(lane == i1[:, None], -1.0, probs)
    v2 = jnp.max(probs2, axis=-1, keepdims=True)
    i2 = jnp.min(jnp.where(probs2 == v2, lane, E), axis=-1)
    tot = v1 + v2
    i1_ref[...] = i1
    i2_ref[...] = i2
    w1_ref[...] = (v1 / tot)[:, 0]
    w2_ref[...] = (v2 / tot)[:, 0]
    c = ((lane == i1[:, None]).astype(jnp.float32)
         + (lane == i2[:, None]).astype(jnp.float32))
    cblk = jnp.sum(c, axis=0, keepdims=True)
    pblk = jnp.sum(probs, axis=0, keepdims=True)

    @pl.when(step == 0)
    def _():
        cnt_ref[...] = cblk
        ps_ref[...] = pblk

    @pl.when(step != 0)
    def _():
        cnt_ref[...] += cblk
        ps_ref[...] += pblk

    @pl.when(step == (N // TBLK) - 1)
    def _():
        aux_ref[...] = (E / (N * 2.0 * N)) * jnp.sum(
            cnt_ref[...] * ps_ref[...], keepdims=True)


_router = pl.pallas_call(
    _router_body,
    grid=(N // TBLK,),
    in_specs=[
        pl.BlockSpec((TBLK, D), lambda i: (i, 0)),
        pl.BlockSpec((D, E), lambda i: (0, 0)),
    ],
    out_specs=[
        pl.BlockSpec((TBLK,), lambda i: (i,)),
        pl.BlockSpec((TBLK,), lambda i: (i,)),
        pl.BlockSpec((TBLK,), lambda i: (i,)),
        pl.BlockSpec((TBLK,), lambda i: (i,)),
        pl.BlockSpec((1, E), lambda i: (0, 0)),
        pl.BlockSpec((1, E), lambda i: (0, 0)),
        pl.BlockSpec((1, 1), lambda i: (0, 0)),
    ],
    out_shape=[
        jax.ShapeDtypeStruct((N,), jnp.int32),
        jax.ShapeDtypeStruct((N,), jnp.int32),
        jax.ShapeDtypeStruct((N,), jnp.float32),
        jax.ShapeDtypeStruct((N,), jnp.float32),
        jax.ShapeDtypeStruct((1, E), jnp.float32),
        jax.ShapeDtypeStruct((1, E), jnp.float32),
        jax.ShapeDtypeStruct((1, 1), jnp.float32),
    ],
)


E2 = E // 2
SLOTS2 = E2 * CAP


def _make_dispatch(ebase):
    @functools.partial(
        pl.kernel,
        out_type=(
            jax.ShapeDtypeStruct((SLOTS2,), jnp.int32),
            jax.ShapeDtypeStruct((E2, CAP), jnp.float32),
            jax.ShapeDtypeStruct((SLOTS2, D), jnp.float32),
        ),
        mesh=_mesh,
        compiler_params=pltpu.CompilerParams(needs_layout_passes=False),
        scratch_types=[
            pltpu.VMEM((N,), jnp.int32),
            pltpu.VMEM((N,), jnp.int32),
            pltpu.VMEM((N,), jnp.float32),
            pltpu.VMEM((N,), jnp.float32),
            pltpu.VMEM((CAP + L,), jnp.int32),
            pltpu.VMEM((CAP + L,), jnp.float32),
            pltpu.VMEM((GCH, D), jnp.float32),
            pltpu.SemaphoreType.DMA,
        ],
        name=f"dispatch_e{ebase}",
    )
    def _disp(i1_hbm, i2_hbm, w1_hbm, w2_hbm, x_hbm,
              sel_hbm, wsel_hbm, xg_hbm,
              i1v, i2v, w1v, w2v, selbuf, wbuf, rowbuf, sem):
        cid = lax.axis_index("c")
        sid = lax.axis_index("s")
        wid = sid * NC + cid
        eg = ebase + wid
        pltpu.sync_copy(i1_hbm, i1v)
        pltpu.sync_copy(i2_hbm, i2v)
        pltpu.sync_copy(w1_hbm, w1v)
        pltpu.sync_copy(w2_hbm, w2v)
        zi = jnp.zeros((L,), jnp.int32)
        zf = jnp.zeros((L,), jnp.float32)
        for j in range((CAP + L) // L):
            selbuf[pl.ds(j * L, L)] = zi
            wbuf[pl.ds(j * L, L)] = zf

        def body(cc, offs):
            a1 = i1v[pl.ds(cc * L, L)]
            a2 = i2v[pl.ds(cc * L, L)]
            m1 = a1 == eg
            m = jnp.logical_or(m1, a2 == eg)
            wv = jnp.where(m1, w1v[pl.ds(cc * L, L)], w2v[pl.ds(cc * L, L)])
            tokv = lax.iota(jnp.int32, L) + cc * L
            ones = jnp.ones((L,), jnp.int32)
            pos = offs + plsc.cumsum(ones, mask=m) - 1

            @pl.when(offs < CAP - 1)
            def _():
                plsc.store_scatter(selbuf, [pos], tokv, mask=m)
                plsc.store_scatter(wbuf, [pos], wv, mask=m)

            pc = plsc.all_reduce_population_count(m)
            return offs + jnp.max(pc)

        cnt = lax.fori_loop(0, N // L, body, jnp.int32(0))
        cnt = jnp.minimum(cnt, CAP - 1)
        lanei = lax.iota(jnp.int32, L)
        wlast = wbuf[pl.ds(CAP - L, L)]
        wbuf[pl.ds(CAP - L, L)] = jnp.where(lanei == L - 1, 0.0, wlast)
        pltpu.sync_copy(selbuf.at[pl.ds(0, CAP)],
                        sel_hbm.at[pl.ds(wid * CAP, CAP)])
        pltpu.sync_copy(wbuf.at[pl.ds(0, CAP)], wsel_hbm.at[wid])

        nch = lax.div(cnt + (GCH - 1), jnp.int32(GCH))

        def gbody(j, carry):
            pltpu.async_copy(
                x_hbm.at[selbuf.at[pl.ds(j * GCH, GCH)]], rowbuf, sem).wait()
            pltpu.sync_copy(rowbuf, xg_hbm.at[pl.ds(wid * CAP + j * GCH, GCH)])
            return carry

        lax.fori_loop(0, nch, gbody, jnp.int32(0))

    return _disp


_dispatch0 = _make_dispatch(0)
_dispatch1 = _make_dispatch(E2)


def _ffn_body(xg_ref, w1_ref, b1_ref, w2_ref, b2_ref, ws_ref, yg_ref):
    h = jnp.dot(xg_ref[...], w1_ref[0], preferred_element_type=jnp.float32)
    h = h + b1_ref[0]
    h = h * lax.logistic(h)
    y = jnp.dot(h, w2_ref[0], preferred_element_type=jnp.float32)
    y = y + b2_ref[0]
    w = ws_ref[0, 0][:, None]
    yg_ref[...] = jnp.where(w > 0.0, y * w, 0.0)


def _make_ffn(ebase):
    return pl.pallas_call(
        _ffn_body,
        grid=(E2,),
        in_specs=[
            pl.BlockSpec((CAP, D), lambda e: (e, 0)),
            pl.BlockSpec((1, D, H), lambda e: (e + ebase, 0, 0)),
            pl.BlockSpec((1, 1, H), lambda e: (e + ebase, 0, 0)),
            pl.BlockSpec((1, H, D), lambda e: (e + ebase, 0, 0)),
            pl.BlockSpec((1, 1, D), lambda e: (e + ebase, 0, 0)),
            pl.BlockSpec((1, 1, CAP), lambda e: (e, 0, 0)),
        ],
        out_specs=pl.BlockSpec((CAP, D), lambda e: (e, 0)),
        out_shape=jax.ShapeDtypeStruct((SLOTS2, D), jnp.float32),
    )


_ffn0 = _make_ffn(0)
_ffn1 = _make_ffn(E2)


TPW = N // NW
SBLK = 2048


@functools.partial(
    pl.kernel,
    out_type=jax.ShapeDtypeStruct((N, D), jnp.float32),
    mesh=_mesh,
    compiler_params=pltpu.CompilerParams(needs_layout_passes=False),
    scratch_types=[
        pltpu.VMEM((TPW,), jnp.int32),
        pltpu.VMEM((TPW,), jnp.int32),
        pltpu.VMEM((TPW,), jnp.int32),
        pltpu.VMEM((TPW,), jnp.int32),
        pltpu.VMEM((TPW,), jnp.int32),
        pltpu.VMEM((SBLK,), jnp.int32),
        pltpu.VMEM((SBLK,), jnp.float32),
        pltpu.VMEM((TPW, D), jnp.float32),
        pltpu.VMEM((TPW, D), jnp.float32),
        pltpu.SemaphoreType.DMA,
    ],
)
def _combine(yg0_hbm, yg1_hbm, sel0_hbm, sel1_hbm, ws0_hbm, ws1_hbm, i1_hbm,
             out_hbm, i1buf, p0a, p0b, p1a, p1b, selblk, wblk,
             rowsA, rowsB, sem):
    cid = lax.axis_index("c")
    sid = lax.axis_index("s")
    wid = sid * NC + cid
    tok0 = wid * TPW
    pltpu.sync_copy(i1_hbm.at[pl.ds(tok0, TPW)], i1buf)
    zslot = jnp.full((L,), CAP - 1, jnp.int32)
    for j in range(TPW // L):
        p0a[pl.ds(j * L, L)] = zslot
        p0b[pl.ds(j * L, L)] = zslot
        p1a[pl.ds(j * L, L)] = zslot
        p1b[pl.ds(j * L, L)] = zslot
    for half, (sel_hbm, ws_hbm, pa, pb) in enumerate(
            ((sel0_hbm, ws0_hbm, p0a, p0b), (sel1_hbm, ws1_hbm, p1a, p1b))):
        for blk in range(SLOTS2 // SBLK):
            pltpu.sync_copy(sel_hbm.at[pl.ds(blk * SBLK, SBLK)], selblk)
            pltpu.sync_copy(ws_hbm.at[pl.ds(blk * SBLK, SBLK)], wblk)

            def scan(cc, carry, blk=blk, half=half, pa=pa, pb=pb):
                base = blk * SBLK + cc * L
                slotv = lax.iota(jnp.int32, L) + base
                tokv = selblk[pl.ds(cc * L, L)]
                wv = wblk[pl.ds(cc * L, L)]
                m = jnp.logical_and(wv > 0.0,
                                    lax.shift_right_logical(tokv, 6) == wid)
                lt = jnp.bitwise_and(tokv, TPW - 1)
                evec = lax.shift_right_logical(slotv, 8) + half * E2
                i1g = plsc.load_gather(i1buf, [lt])
                m1 = jnp.logical_and(m, i1g == evec)
                m2 = jnp.logical_and(m, i1g != evec)
                plsc.store_scatter(pa, [lt], slotv, mask=m1)
                plsc.store_scatter(pb, [lt], slotv, mask=m2)
                return carry

            lax.fori_loop(0, SBLK // L, scan, jnp.int32(0))

    def accrow(rr, carry):
        for k in range(D // L):
            a = rowsA[rr, pl.ds(k * L, L)]
            b = rowsB[rr, pl.ds(k * L, L)]
            rowsA[rr, pl.ds(k * L, L)] = a + b
        return carry

    pltpu.async_copy(yg0_hbm.at[p0a], rowsA, sem).wait()
    pltpu.async_copy(yg0_hbm.at[p0b], rowsB, sem).wait()
    lax.fori_loop(0, TPW, accrow, jnp.int32(0))
    pltpu.async_copy(yg1_hbm.at[p1a], rowsB, sem).wait()
    lax.fori_loop(0, TPW, accrow, jnp.int32(0))
    pltpu.async_copy(yg1_hbm.at[p1b], rowsB, sem).wait()
    lax.fori_loop(0, TPW, accrow, jnp.int32(0))
    pltpu.sync_copy(rowsA, out_hbm.at[pl.ds(tok0, TPW)])


def kernel(x, Wr, W1, b1, W2, b2):
    x2d = x.reshape(N, D)
    i1, i2, w1, w2, _, _, aux = _router(x2d, Wr)
    b1r = b1.reshape(E, 1, H)
    b2r = b2.reshape(E, 1, D)
    sel0, ws0, xg0 = _dispatch0(i1, i2, w1, w2, x2d)
    sel1, ws1, xg1 = _dispatch1(i1, i2, w1, w2, x2d)
    yg0 = _ffn0(xg0, W1, b1r, W2, b2r, ws0.reshape(E2, 1, CAP))
    yg1 = _ffn1(xg1, W1, b1r, W2, b2r, ws1.reshape(E2, 1, CAP))
    out2d = _combine(yg0, yg1, sel0, sel1,
                     ws0.reshape(SLOTS2), ws1.reshape(SLOTS2), i1)
    return out2d.reshape(x.shape), aux[0, 0]


# speedup vs baseline: 1.4079x; 1.4079x over previous
import functools

import jax
import jax.numpy as jnp
from jax import lax
from jax.experimental import pallas as pl
from jax.experimental.pallas import tpu as pltpu
from jax.experimental.pallas import tpu_sc as plsc

N = 2048
D = 768
E = 64
H = 1536
CAP = 256
NC, NS, L = 2, 16, 16
NW = NC * NS
EPW = E // NW
SLOTS = E * CAP
SPW = SLOTS // NW
TBLK = 256
GCH = 64

_mesh = plsc.VectorSubcoreMesh(
    core_axis_name="c", subcore_axis_name="s", num_cores=NC, num_subcores=NS)


def _router_body(x_ref, wr_ref, i1_ref, i2_ref, w1_ref, w2_ref,
                 cnt_ref, ps_ref, aux_ref):
    step = pl.program_id(0)
    logits = jnp.dot(x_ref[...], wr_ref[...],
                     preferred_element_type=jnp.float32)
    m = jnp.max(logits, axis=-1, keepdims=True)
    p = jnp.exp(logits - m)
    probs = p / jnp.sum(p, axis=-1, keepdims=True)
    lane = lax.broadcasted_iota(jnp.int32, (TBLK, E), 1)
    v1 = jnp.max(probs, axis=-1, keepdims=True)
    i1 = jnp.min(jnp.where(probs == v1, lane, E), axis=-1)
    probs2 = jnp.where(lane == i1[:, None], -1.0, probs)
    v2 = jnp.max(probs2, axis=-1, keepdims=True)
    i2 = jnp.min(jnp.where(probs2 == v2, lane, E), axis=-1)
    tot = v1 + v2
    i1_ref[...] = i1
    i2_ref[...] = i2
    w1_ref[...] = (v1 / tot)[:, 0]
    w2_ref[...] = (v2 / tot)[:, 0]
    c = ((lane == i1[:, None]).astype(jnp.float32)
         + (lane == i2[:, None]).astype(jnp.float32))
    cblk = jnp.sum(c, axis=0, keepdims=True)
    pblk = jnp.sum(probs, axis=0, keepdims=True)

    @pl.when(step == 0)
    def _():
        cnt_ref[...] = cblk
        ps_ref[...] = pblk

    @pl.when(step != 0)
    def _():
        cnt_ref[...] += cblk
        ps_ref[...] += pblk

    @pl.when(step == (N // TBLK) - 1)
    def _():
        aux_ref[...] = (E / (N * 2.0 * N)) * jnp.sum(
            cnt_ref[...] * ps_ref[...], keepdims=True)


_router = pl.pallas_call(
    _router_body,
    grid=(N // TBLK,),
    in_specs=[
        pl.BlockSpec((TBLK, D), lambda i: (i, 0)),
        pl.BlockSpec((D, E), lambda i: (0, 0)),
    ],
    out_specs=[
        pl.BlockSpec((TBLK,), lambda i: (i,)),
        pl.BlockSpec((TBLK,), lambda i: (i,)),
        pl.BlockSpec((TBLK,), lambda i: (i,)),
        pl.BlockSpec((TBLK,), lambda i: (i,)),
        pl.BlockSpec((1, E), lambda i: (0, 0)),
        pl.BlockSpec((1, E), lambda i: (0, 0)),
        pl.BlockSpec((1, 1), lambda i: (0, 0)),
    ],
    out_shape=[
        jax.ShapeDtypeStruct((N,), jnp.int32),
        jax.ShapeDtypeStruct((N,), jnp.int32),
        jax.ShapeDtypeStruct((N,), jnp.float32),
        jax.ShapeDtypeStruct((N,), jnp.float32),
        jax.ShapeDtypeStruct((1, E), jnp.float32),
        jax.ShapeDtypeStruct((1, E), jnp.float32),
        jax.ShapeDtypeStruct((1, 1), jnp.float32),
    ],
)


@functools.partial(
    pl.kernel,
    out_type=(
        jax.ShapeDtypeStruct((SLOTS,), jnp.int32),
        jax.ShapeDtypeStruct((E, CAP), jnp.float32),
        jax.ShapeDtypeStruct((SLOTS, D), jnp.float32),
    ),
    mesh=_mesh,
    compiler_params=pltpu.CompilerParams(needs_layout_passes=False),
    scratch_types=[
        pltpu.VMEM((N,), jnp.int32),
        pltpu.VMEM((N,), jnp.int32),
        pltpu.VMEM((N,), jnp.float32),
        pltpu.VMEM((N,), jnp.float32),
        pltpu.VMEM((CAP + L,), jnp.int32),
        pltpu.VMEM((CAP + L,), jnp.float32),
        pltpu.VMEM((GCH, D), jnp.float32),
        pltpu.SemaphoreType.DMA,
    ],
)
def _dispatch(i1_hbm, i2_hbm, w1_hbm, w2_hbm, x_hbm, sel_hbm, wsel_hbm, xg_hbm,
              i1v, i2v, w1v, w2v, selbuf, wbuf, rowbuf, sem):
    cid = lax.axis_index("c")
    sid = lax.axis_index("s")
    wid = sid * NC + cid
    pltpu.sync_copy(i1_hbm, i1v)
    pltpu.sync_copy(i2_hbm, i2v)
    pltpu.sync_copy(w1_hbm, w1v)
    pltpu.sync_copy(w2_hbm, w2v)
    zi = jnp.zeros((L,), jnp.int32)
    zf = jnp.zeros((L,), jnp.float32)
    for ee in range(EPW):
        e = wid * EPW + ee
        for j in range((CAP + L) // L):
            selbuf[pl.ds(j * L, L)] = zi
            wbuf[pl.ds(j * L, L)] = zf

        def body(cc, offs):
            a1 = i1v[pl.ds(cc * L, L)]
            a2 = i2v[pl.ds(cc * L, L)]
            m1 = a1 == e
            m = jnp.logical_or(m1, a2 == e)
            wv = jnp.where(m1, w1v[pl.ds(cc * L, L)], w2v[pl.ds(cc * L, L)])
            tokv = lax.iota(jnp.int32, L) + cc * L

            ones = jnp.ones((L,), jnp.int32)
            pos = offs + plsc.cumsum(ones, mask=m) - 1

            @pl.when(offs < CAP - 1)
            def _():
                plsc.store_scatter(selbuf, [pos], tokv, mask=m)
                plsc.store_scatter(wbuf, [pos], wv, mask=m)

            pc = plsc.all_reduce_population_count(m)
            return offs + jnp.max(pc)

        cnt = lax.fori_loop(0, N // L, body, jnp.int32(0))
        cnt = jnp.minimum(cnt, CAP - 1)
        lanei = lax.iota(jnp.int32, L)
        wlast = wbuf[pl.ds(CAP - L, L)]
        wbuf[pl.ds(CAP - L, L)] = jnp.where(lanei == L - 1, 0.0, wlast)
        pltpu.sync_copy(selbuf.at[pl.ds(0, CAP)], sel_hbm.at[pl.ds(e * CAP, CAP)])
        pltpu.sync_copy(wbuf.at[pl.ds(0, CAP)], wsel_hbm.at[e])

        nch = lax.div(cnt + (GCH - 1), jnp.int32(GCH))

        def gbody(j, carry):
            pltpu.async_copy(
                x_hbm.at[selbuf.at[pl.ds(j * GCH, GCH)]], rowbuf, sem).wait()
            pltpu.sync_copy(rowbuf, xg_hbm.at[pl.ds(e * CAP + j * GCH, GCH)])
            return carry

        lax.fori_loop(0, nch, gbody, jnp.int32(0))


def _ffn_body(xg_ref, w1_ref, b1_ref, w2_ref, b2_ref, ws_ref, yg_ref):
    h = jnp.dot(xg_ref[...], w1_ref[0], preferred_element_type=jnp.float32)
    h = h + b1_ref[0]
    h = h * lax.logistic(h)
    y = jnp.dot(h, w2_ref[0], preferred_element_type=jnp.float32)
    y = y + b2_ref[0]
    w = ws_ref[0, 0][:, None]
    yg_ref[...] = jnp.where(w > 0.0, y * w, 0.0)


_ffn = pl.pallas_call(
    _ffn_body,
    grid=(E,),
    in_specs=[
        pl.BlockSpec((CAP, D), lambda e: (e, 0)),
        pl.BlockSpec((1, D, H), lambda e: (e, 0, 0)),
        pl.BlockSpec((1, 1, H), lambda e: (e, 0, 0)),
        pl.BlockSpec((1, H, D), lambda e: (e, 0, 0)),
        pl.BlockSpec((1, 1, D), lambda e: (e, 0, 0)),
        pl.BlockSpec((1, 1, CAP), lambda e: (e, 0, 0)),
    ],
    out_specs=pl.BlockSpec((CAP, D), lambda e: (e, 0)),
    out_shape=jax.ShapeDtypeStruct((SLOTS, D), jnp.float32),
)


TPW = N // NW
SBLK = 2048


@functools.partial(
    pl.kernel,
    out_type=jax.ShapeDtypeStruct((N, D), jnp.float32),
    mesh=_mesh,
    compiler_params=pltpu.CompilerParams(needs_layout_passes=False),
    scratch_types=[
        pltpu.VMEM((TPW,), jnp.int32),
        pltpu.VMEM((TPW,), jnp.int32),
        pltpu.VMEM((TPW,), jnp.int32),
        pltpu.VMEM((SBLK,), jnp.int32),
        pltpu.VMEM((SBLK,), jnp.float32),
        pltpu.VMEM((TPW, D), jnp.float32),
        pltpu.VMEM((TPW, D), jnp.float32),
        pltpu.SemaphoreType.DMA,
    ],
)
def _combine(yg_hbm, sel_hbm, wsel_hbm, i1_hbm, out_hbm,
             i1buf, posb1, posb2, selblk, wblk, rowsA, rowsB, sem):
    cid = lax.axis_index("c")
    sid = lax.axis_index("s")
    wid = sid * NC + cid
    tok0 = wid * TPW
    pltpu.sync_copy(i1_hbm.at[pl.ds(tok0, TPW)], i1buf)
    zslot = jnp.full((L,), CAP - 1, jnp.int32)
    for j in range(TPW // L):
        posb1[pl.ds(j * L, L)] = zslot
        posb2[pl.ds(j * L, L)] = zslot
    for blk in range(SLOTS // SBLK):
        pltpu.sync_copy(sel_hbm.at[pl.ds(blk * SBLK, SBLK)], selblk)
        pltpu.sync_copy(wsel_hbm.at[pl.ds(blk * SBLK, SBLK)], wblk)

        def scan(cc, carry):
            base = blk * SBLK + cc * L
            slotv = lax.iota(jnp.int32, L) + base
            tokv = selblk[pl.ds(cc * L, L)]
            wv = wblk[pl.ds(cc * L, L)]
            m = jnp.logical_and(wv > 0.0,
                                lax.shift_right_logical(tokv, 6) == wid)
            lt = jnp.bitwise_and(tokv, TPW - 1)
            evec = lax.shift_right_logical(slotv, 8)
            i1g = plsc.load_gather(i1buf, [lt])
            m1 = jnp.logical_and(m, i1g == evec)
            m2 = jnp.logical_and(m, i1g != evec)
            plsc.store_scatter(posb1, [lt], slotv, mask=m1)
            plsc.store_scatter(posb2, [lt], slotv, mask=m2)
            return carry

        lax.fori_loop(0, SBLK // L, scan, jnp.int32(0))
    pltpu.async_copy(yg_hbm.at[posb1], rowsA, sem).wait()
    pltpu.async_copy(yg_hbm.at[posb2], rowsB, sem).wait()

    def addrow(rr, carry):
        for k in range(D // L):
            a = rowsA[rr, pl.ds(k * L, L)]
            b = rowsB[rr, pl.ds(k * L, L)]
            rowsA[rr, pl.ds(k * L, L)] = a + b
        return carry

    lax.fori_loop(0, TPW, addrow, jnp.int32(0))
    pltpu.sync_copy(rowsA, out_hbm.at[pl.ds(tok0, TPW)])


def kernel(x, Wr, W1, b1, W2, b2):
    x2d = x.reshape(N, D)
    i1, i2, w1, w2, _, _, aux = _router(x2d, Wr)
    sel, wsel, xg = _dispatch(i1, i2, w1, w2, x2d)
    yg = _ffn(xg, W1, b1.reshape(E, 1, H), W2, b2.reshape(E, 1, D),
              wsel.reshape(E, 1, CAP))
    out2d = _combine(yg, sel, wsel.reshape(SLOTS), i1)
    return out2d.reshape(x.shape), aux[0, 0]
